# baseline (speedup 1.0000x reference)
import functools

import jax
import jax.numpy as jnp
from jax import lax
from jax.experimental import pallas as pl
from jax.experimental.pallas import tpu as pltpu
from jax.experimental.pallas import tpu_sc as plsc

B = 64
S = 512
H = 768
V16 = H // 16
NC = 2
NS = 16
NW = NC * NS
SLAB = S // NW
TPW = B * SLAB
NB = 4
G = NB * SLAB
NCHUNK = TPW // G
EPS = 1e-05

_mesh = plsc.VectorSubcoreMesh(
    core_axis_name="c", subcore_axis_name="s", num_cores=NC, num_subcores=NS
)


def _hsum(v):
    lanes = lax.iota(jnp.int32, 16)
    for k in (1, 2, 4, 8):
        v = v + jnp.take_along_axis(v, lanes ^ k, axis=0,
                                    mode="promise_in_bounds")
    return v


def _body(ids_h, ttf_h, word_h, base_h, dseg_h, gam_h, bet_h, out_h,
          ids_v, ttf_v, base_v, dseg_v, gam_v, bet_v, rows0, rows1,
          sem0, sem1):
    cid = lax.axis_index("c")
    sid = lax.axis_index("s")
    wid = sid * NC + cid

    pltpu.sync_copy(ids_h.at[wid], ids_v)
    pltpu.sync_copy(ttf_h.at[wid], ttf_v)
    pltpu.sync_copy(base_h.at[pl.ds(wid * SLAB, SLAB)], base_v)
    pltpu.sync_copy(dseg_h, dseg_v)
    pltpu.sync_copy(gam_h, gam_v)
    pltpu.sync_copy(bet_h, bet_v)

    def start_gather(c, rows, sem):
        idx = ids_v.at[pl.ds(c * G, G)]
        return pltpu.async_copy(word_h.at[idx], rows, sem)

    def wait_gather(rows, sem):
        idx = ids_v.at[pl.ds(0, G)]
        pltpu.make_async_copy(word_h.at[idx], rows, sem).wait()

    def process(c, rows):
        coff = c * G

        @plsc.parallel_loop(0, SLAB)
        def grp(i):
            lanes_i = jnp.full((16,), i, jnp.int32)
            ttf = [
                jnp.take_along_axis(
                    ttf_v[pl.ds(coff + k * SLAB, 16)], lanes_i, axis=0,
                    mode="promise_in_bounds")
                for k in range(NB)
            ]
            acc_s = [jnp.zeros((16,), jnp.float32) for _ in range(NB)]
            acc_q = [jnp.zeros((16,), jnp.float32) for _ in range(NB)]
            for j in range(V16):
                sl = pl.ds(j * 16, 16)
                bse = base_v[i, sl]
                dsg = dseg_v[sl]
                for k in range(NB):
                    v = rows[k * SLAB + i, sl] + (bse + ttf[k] * dsg)
                    rows[k * SLAB + i, sl] = v
                    acc_s[k] = acc_s[k] + v
                    acc_q[k] = acc_q[k] + v * v
            mean = [None] * NB
            rstd = [None] * NB
            for k in range(NB):
                m = _hsum(acc_s[k])[0] * (1.0 / H)
                var = _hsum(acc_q[k])[0] * (1.0 / H) - m * m
                x = var + EPS
                b = lax.bitcast_convert_type(x, jnp.int32)
                b = jnp.int32(0x5F3759DF) - (b >> 1)
                ys = lax.bitcast_convert_type(b, jnp.float32)
                for _ in range(3):
                    ys = ys * (1.5 - 0.5 * x * ys * ys)
                mean[k] = jnp.full((16,), m, dtype=jnp.float32)
                rstd[k] = jnp.full((16,), ys, dtype=jnp.float32)
            for j in range(V16):
                sl = pl.ds(j * 16, 16)
                g = gam_v[sl]
                bb = bet_v[sl]
                for k in range(NB):
                    v = rows[k * SLAB + i, sl]
                    rows[k * SLAB + i, sl] = (v - mean[k]) * rstd[k] * g + bb
            return ()

        for k in range(NB):
            orow = (NB * c + k) * S + wid * SLAB
            pltpu.sync_copy(rows.at[pl.ds(k * SLAB, SLAB)],
                            out_h.at[pl.ds(orow, SLAB)])

    start_gather(0, rows0, sem0)
    start_gather(1, rows1, sem1)

    def outer(p, carry):
        c0 = 2 * p
        wait_gather(rows0, sem0)
        process(c0, rows0)

        @pl.when(c0 + 2 < NCHUNK)
        def _():
            start_gather(c0 + 2, rows0, sem0)

        wait_gather(rows1, sem1)
        process(c0 + 1, rows1)

        @pl.when(c0 + 3 < NCHUNK)
        def _():
            start_gather(c0 + 3, rows1, sem1)

        return carry

    lax.fori_loop(0, NCHUNK // 2, outer, 0)


_emb_ln = functools.partial(
    pl.kernel,
    out_type=jax.ShapeDtypeStruct((B * S, H), jnp.float32),
    mesh=_mesh,
    scratch_types=[
        pltpu.VMEM((TPW,), jnp.int32),
        pltpu.VMEM((TPW,), jnp.float32),
        pltpu.VMEM((SLAB, H), jnp.float32),
        pltpu.VMEM((H,), jnp.float32),
        pltpu.VMEM((H,), jnp.float32),
        pltpu.VMEM((H,), jnp.float32),
        pltpu.VMEM((G, H), jnp.float32),
        pltpu.VMEM((G, H), jnp.float32),
        pltpu.SemaphoreType.DMA,
        pltpu.SemaphoreType.DMA,
    ],
)(_body)


def kernel(input_ids, token_type_ids, word_embedding, segment_embedding,
           position_embedding, ln_gamma, ln_beta):
    ids = input_ids.astype(jnp.int32)
    ids_w = ids.reshape(B, NW, SLAB).transpose(1, 0, 2).reshape(NW, TPW)
    ttf_w = (token_type_ids.astype(jnp.float32)
             .reshape(B, NW, SLAB).transpose(1, 0, 2).reshape(NW, TPW))
    base = position_embedding + segment_embedding[0][None, :]
    dseg = segment_embedding[1] - segment_embedding[0]
    out = _emb_ln(ids_w, ttf_w, word_embedding, base, dseg, ln_gamma, ln_beta)
    return out.reshape(B, S, H)


# speedup vs baseline: 4.2366x; 4.2366x over previous
import functools

import jax
import jax.numpy as jnp
from jax import lax
from jax.experimental import pallas as pl
from jax.experimental.pallas import tpu as pltpu
from jax.experimental.pallas import tpu_sc as plsc

B = 64
S = 512
H = 768
V16 = H // 16
NC = 2
NS = 16
NW = NC * NS
SLAB = S // NW
TPW = B * SLAB
NB = 4
G = NB * SLAB
NCHUNK = TPW // G
EPS = 1e-05

_mesh = plsc.VectorSubcoreMesh(
    core_axis_name="c", subcore_axis_name="s", num_cores=NC, num_subcores=NS
)


def _hsum(v):
    lanes = lax.iota(jnp.int32, 16)
    for k in (1, 2, 4, 8):
        v = v + jnp.take_along_axis(v, lanes ^ k, axis=0,
                                    mode="promise_in_bounds")
    return v


def _body(ids_h, tti_h, word_h, comb_h, gam_h, bet_h, out_h,
          ids_v, tti_v, comb_v, gam_v, bet_v, rows0, rows1,
          sem0, sem1):
    cid = lax.axis_index("c")
    sid = lax.axis_index("s")
    wid = sid * NC + cid

    pltpu.sync_copy(ids_h.at[wid], ids_v)
    pltpu.sync_copy(tti_h.at[wid], tti_v)
    pltpu.sync_copy(comb_h.at[pl.ds(wid * SLAB, SLAB)],
                    comb_v.at[pl.ds(0, SLAB)])
    pltpu.sync_copy(comb_h.at[pl.ds(S + wid * SLAB, SLAB)],
                    comb_v.at[pl.ds(SLAB, SLAB)])
    pltpu.sync_copy(gam_h, gam_v)
    pltpu.sync_copy(bet_h, bet_v)

    def start_gather(c, rows, sem):
        idx = ids_v.at[pl.ds(c * G, G)]
        return pltpu.async_copy(word_h.at[idx], rows, sem)

    def wait_gather(rows, sem):
        idx = ids_v.at[pl.ds(0, G)]
        pltpu.make_async_copy(word_h.at[idx], rows, sem).wait()

    def process(c, rows):
        coff = c * G

        @plsc.parallel_loop(0, G)
        def tok(t):
            s_loc = t & (SLAB - 1)
            tt16 = tti_v[pl.ds(coff + t - s_loc, 16)]
            lanes = lax.iota(jnp.int32, 16)
            sel = jnp.where(lanes == s_loc, tt16, 0)
            itt = _hsum(sel)[0]
            row = itt + s_loc
            acc_s = jnp.zeros((16,), jnp.float32)
            acc_q = jnp.zeros((16,), jnp.float32)
            for j in range(V16):
                sl = pl.ds(j * 16, 16)
                v = rows[t, sl] + comb_v[row, sl]
                rows[t, sl] = v
                acc_s = acc_s + v
                acc_q = acc_q + v * v
            mean = _hsum(acc_s)[0] * (1.0 / H)
            var = _hsum(acc_q)[0] * (1.0 / H) - mean * mean
            x = var + EPS
            i = lax.bitcast_convert_type(x, jnp.int32)
            i = jnp.int32(0x5F3759DF) - (i >> 1)
            ys = lax.bitcast_convert_type(i, jnp.float32)
            for _ in range(3):
                ys = ys * (1.5 - 0.5 * x * ys * ys)
            mean_v = jnp.full((16,), mean, dtype=jnp.float32)
            y = jnp.full((16,), ys, dtype=jnp.float32)
            for j in range(V16):
                sl = pl.ds(j * 16, 16)
                v = rows[t, sl]
                rows[t, sl] = (v - mean_v) * y * gam_v[sl] + bet_v[sl]
            return ()

        for k in range(NB):
            orow = (NB * c + k) * S + wid * SLAB
            pltpu.sync_copy(rows.at[pl.ds(k * SLAB, SLAB)],
                            out_h.at[pl.ds(orow, SLAB)])

    start_gather(0, rows0, sem0)
    start_gather(1, rows1, sem1)

    def outer(p, carry):
        c0 = 2 * p
        wait_gather(rows0, sem0)
        process(c0, rows0)

        @pl.when(c0 + 2 < NCHUNK)
        def _():
            start_gather(c0 + 2, rows0, sem0)

        wait_gather(rows1, sem1)
        process(c0 + 1, rows1)

        @pl.when(c0 + 3 < NCHUNK)
        def _():
            start_gather(c0 + 3, rows1, sem1)

        return carry

    lax.fori_loop(0, NCHUNK // 2, outer, 0)


_emb_ln = functools.partial(
    pl.kernel,
    out_type=jax.ShapeDtypeStruct((B * S, H), jnp.float32),
    mesh=_mesh,
    scratch_types=[
        pltpu.VMEM((TPW,), jnp.int32),
        pltpu.VMEM((TPW,), jnp.int32),
        pltpu.VMEM((2 * SLAB, H), jnp.float32),
        pltpu.VMEM((H,), jnp.float32),
        pltpu.VMEM((H,), jnp.float32),
        pltpu.VMEM((G, H), jnp.float32),
        pltpu.VMEM((G, H), jnp.float32),
        pltpu.SemaphoreType.DMA,
        pltpu.SemaphoreType.DMA,
    ],
)(_body)


def kernel(input_ids, token_type_ids, word_embedding, segment_embedding,
           position_embedding, ln_gamma, ln_beta):
    ids = input_ids.astype(jnp.int32)
    ids_w = ids.reshape(B, NW, SLAB).transpose(1, 0, 2).reshape(NW, TPW)
    tti_w = (token_type_ids.astype(jnp.int32) * SLAB
             ).reshape(B, NW, SLAB).transpose(1, 0, 2).reshape(NW, TPW)
    comb = (position_embedding[None, :, :]
            + segment_embedding[:, None, :]).reshape(2 * S, H)
    out = _emb_ln(ids_w, tti_w, word_embedding, comb, ln_gamma, ln_beta)
    return out.reshape(B, S, H)
